# baseline (speedup 1.0000x reference)
import dataclasses
import functools

import jax
import jax.numpy as jnp
from jax import lax
from jax.experimental import pallas as pl
from jax.experimental.pallas import tpu as pltpu
from jax.experimental.pallas import tpu_sc as plsc

IGNORE_INDEX = 255
L_MAIN, L_AUX, L_AFF, L_DIST, L_BDY = 1.0, 1.0, 0.1, 0.1, 0.5


def _f8_to_bf16(w):
    f8 = plsc.bitcast(w, jnp.float8_e4m3fn)
    return plsc.unpack(f8, format=plsc.PackFormat.INTERLEAVED,
                       preferred_element_type=jnp.bfloat16)


def _sc_pair_d2(tab_packed, kidx_nat):
    BN, W2 = tab_packed.shape
    C2 = W2 // 2
    BK, N = kidx_nat.shape
    B = BN // N
    K = BK // B
    P = BN * K
    L = 16
    NSUB = 32
    PPS = P // NSUB
    TPS = BN // NSUB
    Wp = 256
    NW = PPS // Wp
    TW = Wp // K
    CH = C2 // L

    mesh = plsc.VectorSubcoreMesh(core_axis_name="core", subcore_axis_name="subcore")
    cp = pltpu.CompilerParams()
    if "needs_layout_passes" in pltpu.CompilerParams.__dataclass_fields__:
        cp = dataclasses.replace(cp, needs_layout_passes=False)

    @functools.partial(
        pl.kernel,
        out_type=(jax.ShapeDtypeStruct((P // 128, 128), jnp.float32),
                  jax.ShapeDtypeStruct((P // 128, 128), jnp.float32)),
        mesh=mesh,
        compiler_params=cp,
        scratch_types=[
            pltpu.VMEM((8, TPS), jnp.int32),
            pltpu.VMEM((2, 8, TW, W2), jnp.int32),
            pltpu.VMEM((2, TW, W2), jnp.int32),
            pltpu.VMEM((PPS // 128, 128), jnp.float32),
            pltpu.VMEM((PPS // 128, 128), jnp.float32),
            pltpu.SemaphoreType.DMA((2,)),
            pltpu.SemaphoreType.DMA((2,)),
        ],
    )
    def sck(tab_hbm, idx_hbm, drf_hbm, djf_hbm,
            idx_all, nbr, ctr, drf_all, djf_all, s_g, s_c):
        wid = lax.axis_index("subcore") * 2 + lax.axis_index("core")
        pbase = wid * PPS
        tbase = wid * TPS
        bat = wid // (NSUB // B)
        n0 = (wid % (NSUB // B)) * TPS
        for k in range(K):
            pltpu.sync_copy(idx_hbm.at[bat * K + k, pl.ds(n0, TPS)],
                            idx_all.at[k])
        boff = bat * N

        @pl.loop(0, TPS // L)
        def _(c):
            for k in range(K):
                s = pl.ds(c * L, L)
                idx_all[k, s] = idx_all[k, s] + boff

        def dmas(w, b):
            t0 = tbase + w * TW
            out = [pltpu.make_async_copy(tab_hbm.at[idx_all.at[k, pl.ds(w * TW, TW)]],
                                         nbr.at[b, k], s_g.at[b])
                   for k in range(K)]
            out.append(pltpu.make_async_copy(tab_hbm.at[pl.ds(t0, TW)],
                                             ctr.at[b], s_c.at[b]))
            return out

        def issue(w, b):
            for d in dmas(w, b):
                d.start()

        def wait(w, b):
            for d in dmas(w, b):
                d.wait()

        def compute(w, b):
            obase = w * Wp
            lanes = lax.iota(jnp.int32, L)

            def gbody(g, carry):
                vrf = jnp.zeros((L,), jnp.float32)
                vjf = jnp.zeros((L,), jnp.float32)
                for jt in range(L // K):
                    t = g * (L // K) + jt
                    crf, cjf = [], []
                    for cc in range(CH):
                        crf.extend(_f8_to_bf16(ctr[b, t, pl.ds(cc * L, L)]))
                        cjf.extend(_f8_to_bf16(ctr[b, t, pl.ds(C2 + cc * L, L)]))
                    for k in range(K):
                        j = jt * K + k
                        tl = g * (L // K) + jt
                        arf = jnp.zeros((L,), jnp.float32)
                        ajf = jnp.zeros((L,), jnp.float32)
                        for cc in range(0, CH, 2):
                            sqs = []
                            for (ctab, off) in ((crf, 0), (cjf, C2)):
                                n0, n1 = _f8_to_bf16(nbr[b, k, tl, pl.ds(off + cc * L, L)])
                                n2, n3 = _f8_to_bf16(nbr[b, k, tl, pl.ds(off + (cc + 1) * L, L)])
                                d0 = ctab[2 * cc] - n0
                                d1 = ctab[2 * cc + 1] - n1
                                d2 = ctab[2 * cc + 2] - n2
                                d3 = ctab[2 * cc + 3] - n3
                                sqs.append((d0 * d0 + d1 * d1) + (d2 * d2 + d3 * d3))
                            sq0, sq1 = plsc.unpack(sqs[0], format=plsc.PackFormat.INTERLEAVED)
                            arf = arf + sq0 + sq1
                            sq2, sq3 = plsc.unpack(sqs[1], format=plsc.PackFormat.INTERLEAVED)
                            ajf = ajf + sq2 + sq3
                        vrf = jnp.where(lanes == j, jnp.sum(arf), vrf)
                        vjf = jnp.where(lanes == j, jnp.sum(ajf), vjf)
                flat = obase + g * L
                row = lax.shift_right_logical(flat, 7)
                col = jnp.bitwise_and(flat, 127)
                drf_all[row, pl.ds(col, L)] = vrf
                djf_all[row, pl.ds(col, L)] = vjf
                return carry

            lax.fori_loop(0, Wp // L, gbody, 0)

        issue(0, 0)

        @pl.loop(0, NW, step=2)
        def _(w):
            for b in (0, 1):
                we = w + b

                @pl.when(we + 1 < NW)
                def _issue_next(we=we, b=b):
                    issue(we + 1, b ^ 1)

                wait(we, b)
                compute(we, b)

        orow = pl.multiple_of(pbase // 128, PPS // 128)
        pltpu.sync_copy(drf_all, drf_hbm.at[pl.ds(orow, PPS // 128), :])
        pltpu.sync_copy(djf_all, djf_hbm.at[pl.ds(orow, PPS // 128), :])

    return sck(tab_packed, kidx_nat)


def _tc_pack(rf, jf):
    BN, C = rf.shape
    C2 = C // 4
    R = 4096
    G = BN // R

    def body(rf_r, jf_r, o_r):
        for src, off in ((rf_r, 0), (jf_r, C2)):
            u = lax.bitcast_convert_type(
                src[...].astype(jnp.float8_e4m3fn), jnp.uint8).astype(jnp.int32)
            o_r[:, off:off + C2] = (u[:, :C2]
                                    | lax.shift_left(u[:, C2:2 * C2], 8)
                                    | lax.shift_left(u[:, 2 * C2:3 * C2], 16)
                                    | lax.shift_left(u[:, 3 * C2:], 24))

    return pl.pallas_call(
        body,
        grid=(G,),
        in_specs=[pl.BlockSpec((R, C), lambda i: (i, 0)),
                  pl.BlockSpec((R, C), lambda i: (i, 0))],
        out_specs=pl.BlockSpec((R, 2 * C2), lambda i: (i, 0)),
        out_shape=jax.ShapeDtypeStruct((BN, 2 * C2), jnp.int32),
    )(rf, jf)


def _tc_dense(rlog, alog, tgt2d, feat, prot):
    BN, NC = rlog.shape
    C = feat.shape[1]
    R = 2048
    G = BN // R

    def body(rlog_r, alog_r, tgt_r, feat_r, prot_r, out_r, acc):
        i = pl.program_id(0)

        @pl.when(i == 0)
        def _init():
            for j in range(8):
                acc[j] = 0.0

        tgt = tgt_r[...]
        valid = tgt != IGNORE_INDEX
        tgt0 = jnp.where(valid, tgt, 0)
        iota = lax.broadcasted_iota(jnp.int32, (R, NC), 1)
        onehot = iota == tgt0

        def ce_sum(lg):
            m = jnp.max(lg, axis=1, keepdims=True)
            l = lg - m
            lse = jnp.log(jnp.sum(jnp.exp(l), axis=1, keepdims=True))
            ltgt = jnp.sum(jnp.where(onehot, l, 0.0), axis=1, keepdims=True)
            return jnp.sum(jnp.where(valid, lse - ltgt, 0.0))

        s_main = ce_sum(rlog_r[...])
        s_aux = ce_sum(alog_r[...])
        n_valid = jnp.sum(valid.astype(jnp.float32))

        f = feat_r[...]
        p = prot_r[...]
        pn = p / jnp.maximum(jnp.sqrt(jnp.sum(p * p, axis=1, keepdims=True)), 1e-12)
        sim = lax.dot_general(f, pn, (((1,), (1,)), ((), ())),
                              preferred_element_type=jnp.float32)
        fnorm = jnp.maximum(jnp.sqrt(jnp.sum(f * f, axis=1, keepdims=True)), 1e-12)
        tsim = jnp.sum(jnp.where(onehot, sim, 0.0), axis=1, keepdims=True) / fnorm
        s_dist = jnp.sum(jnp.where(valid, 1.0 - tsim, 0.0))

        acc[0] += s_main
        acc[1] += s_aux
        acc[2] += s_dist
        acc[3] += n_valid
        for j in range(8):
            out_r[0, j] = acc[j]

    return pl.pallas_call(
        body,
        grid=(G,),
        in_specs=[
            pl.BlockSpec((R, NC), lambda i: (i, 0)),
            pl.BlockSpec((R, NC), lambda i: (i, 0)),
            pl.BlockSpec((R, 1), lambda i: (i, 0)),
            pl.BlockSpec((R, C), lambda i: (i, 0)),
            pl.BlockSpec((NC, C), lambda i: (0, 0)),
        ],
        out_specs=pl.BlockSpec((1, 8), lambda i: (0, 0),
                               memory_space=pltpu.SMEM),
        out_shape=jax.ShapeDtypeStruct((1, 8), jnp.float32),
        scratch_shapes=[pltpu.SMEM((8,), jnp.float32)],
    )(rlog, alog, tgt2d, feat, prot)


def _tc_combine(sums, aff, bdy, drf, djf, C, K, BN):
    PR = aff.shape[0]
    TPR = 128 // K
    R = 256
    G = PR // R

    def body(sums_r, aff_r, bdy_r, drf_r, djf_r, out_r, acc):
        i = pl.program_id(0)

        @pl.when(i == 0)
        def _init():
            for j in range(4):
                acc[j] = 0.0

        w = jnp.maximum(aff_r[...] - 0.5, 0.0)
        acc[0] += jnp.sum(w * drf_r[...])
        acc[1] += jnp.sum(w)

        lane = lax.broadcasted_iota(jnp.int32, (128, TPR), 0)
        tok = lax.broadcasted_iota(jnp.int32, (128, TPR), 1)
        m = (lane // K == tok).astype(jnp.float32)
        es = lax.dot_general(jnp.sqrt(djf_r[...]), m, (((1,), (0,)), ((), ())),
                             preferred_element_type=jnp.float32) / K
        tb = jax.nn.sigmoid((es - 0.15) * 20.0)
        x = bdy_r[...]
        bce = jnp.maximum(x, 0.0) - x * tb + jnp.log1p(jnp.exp(-jnp.abs(x)))
        acc[2] += jnp.sum(bce)

        denom = jnp.maximum(sums_r[0, 3], 1.0)
        loss = (L_MAIN * sums_r[0, 0] + L_AUX * sums_r[0, 1]) / denom
        loss += L_AFF * (acc[0] / (C ** 0.5)) / (acc[1] + 0.0001)
        loss += L_DIST * sums_r[0, 2] / denom
        loss += L_BDY * acc[2] / BN
        out_r[0, 0] = loss

    return pl.pallas_call(
        body,
        grid=(G,),
        in_specs=[
            pl.BlockSpec((1, 8), lambda i: (0, 0), memory_space=pltpu.SMEM),
            pl.BlockSpec((R, 128), lambda i: (i, 0)),
            pl.BlockSpec((R, TPR), lambda i: (i, 0)),
            pl.BlockSpec((R, 128), lambda i: (i, 0)),
            pl.BlockSpec((R, 128), lambda i: (i, 0)),
        ],
        out_specs=pl.BlockSpec((1, 1), lambda i: (0, 0),
                               memory_space=pltpu.SMEM),
        out_shape=jax.ShapeDtypeStruct((1, 1), jnp.float32),
        scratch_shapes=[pltpu.SMEM((4,), jnp.float32)],
    )(sums, aff, bdy, drf, djf)


def kernel(refined_logits, aux_logits, refined_feat, affinity, prototypes,
           input_jafar_feat, bdy_logits, target, k_idx):
    B, N, K = k_idx.shape
    C = refined_feat.shape[-1]
    BN = B * N
    P = BN * K

    kidx_nat = k_idx.transpose(0, 2, 1).reshape(B * K, N)
    rf_flat = refined_feat.reshape(BN, C)
    jf_flat = input_jafar_feat.reshape(BN, C)

    tab_p = _tc_pack(rf_flat, jf_flat)
    d_rf, d_jf = _sc_pair_d2(tab_p, kidx_nat)

    sums = _tc_dense(refined_logits, aux_logits, target.reshape(BN, 1),
                     rf_flat, prototypes)
    out = _tc_combine(sums, affinity.reshape(P // 128, 128),
                      bdy_logits.reshape(BN // 16, 16), d_rf, d_jf, C, K, BN)
    return out[0, 0]


# speedup vs baseline: 1.0210x; 1.0210x over previous
import dataclasses
import functools

import jax
import jax.numpy as jnp
from jax import lax
from jax.experimental import pallas as pl
from jax.experimental.pallas import tpu as pltpu
from jax.experimental.pallas import tpu_sc as plsc

IGNORE_INDEX = 255
L_MAIN, L_AUX, L_AFF, L_DIST, L_BDY = 1.0, 1.0, 0.1, 0.1, 0.5


def _f8_to_bf16(w):
    f8 = plsc.bitcast(w, jnp.float8_e4m3fn)
    return plsc.unpack(f8, format=plsc.PackFormat.INTERLEAVED,
                       preferred_element_type=jnp.bfloat16)


def _sc_pair_d2(tab_packed, kidx_nat):
    BN, W2 = tab_packed.shape
    C2 = W2 // 2
    BK, N = kidx_nat.shape
    B = BN // N
    K = BK // B
    P = BN * K
    L = 16
    NSUB = 32
    PPS = P // NSUB
    TPS = BN // NSUB
    Wp = 128
    NW = PPS // Wp
    TW = Wp // K
    CH = C2 // L

    mesh = plsc.VectorSubcoreMesh(core_axis_name="core", subcore_axis_name="subcore")
    cp = pltpu.CompilerParams()
    if "needs_layout_passes" in pltpu.CompilerParams.__dataclass_fields__:
        cp = dataclasses.replace(cp, needs_layout_passes=False)

    @functools.partial(
        pl.kernel,
        out_type=(jax.ShapeDtypeStruct((P // 128, 128), jnp.float32),
                  jax.ShapeDtypeStruct((P // 128, 128), jnp.float32)),
        mesh=mesh,
        compiler_params=cp,
        scratch_types=[
            pltpu.VMEM((8, TPS), jnp.int32),
            pltpu.VMEM((2, 8, TW, W2), jnp.int32),
            pltpu.VMEM((2, TW, W2), jnp.int32),
            pltpu.VMEM((PPS // 128, 128), jnp.float32),
            pltpu.VMEM((PPS // 128, 128), jnp.float32),
            pltpu.SemaphoreType.DMA((2,)),
            pltpu.SemaphoreType.DMA((2,)),
        ],
    )
    def sck(tab_hbm, idx_hbm, drf_hbm, djf_hbm,
            idx_all, nbr, ctr, drf_all, djf_all, s_g, s_c):
        wid = lax.axis_index("subcore") * 2 + lax.axis_index("core")
        pbase = wid * PPS
        tbase = wid * TPS
        bat = wid // (NSUB // B)
        n0 = (wid % (NSUB // B)) * TPS
        for k in range(K):
            pltpu.sync_copy(idx_hbm.at[bat * K + k, pl.ds(n0, TPS)],
                            idx_all.at[k])
        boff = bat * N

        @pl.loop(0, TPS // L)
        def _(c):
            for k in range(K):
                s = pl.ds(c * L, L)
                idx_all[k, s] = idx_all[k, s] + boff

        def dmas(w, b):
            t0 = tbase + w * TW
            out = [pltpu.make_async_copy(tab_hbm.at[idx_all.at[k, pl.ds(w * TW, TW)]],
                                         nbr.at[b, k], s_g.at[b])
                   for k in range(K)]
            out.append(pltpu.make_async_copy(tab_hbm.at[pl.ds(t0, TW)],
                                             ctr.at[b], s_c.at[b]))
            return out

        def issue(w, b):
            for d in dmas(w, b):
                d.start()

        def wait(w, b):
            for d in dmas(w, b):
                d.wait()

        def compute(w, b):
            obase = w * Wp
            lanes = lax.iota(jnp.int32, L)

            def gbody(g, carry):
                vrf = jnp.zeros((L,), jnp.float32)
                vjf = jnp.zeros((L,), jnp.float32)
                for jt in range(L // K):
                    t = g * (L // K) + jt
                    crf, cjf = [], []
                    for cc in range(CH):
                        crf.extend(_f8_to_bf16(ctr[b, t, pl.ds(cc * L, L)]))
                        cjf.extend(_f8_to_bf16(ctr[b, t, pl.ds(C2 + cc * L, L)]))
                    for k in range(K):
                        j = jt * K + k
                        tl = g * (L // K) + jt
                        arf = jnp.zeros((L,), jnp.float32)
                        ajf = jnp.zeros((L,), jnp.float32)
                        for cc in range(0, CH, 2):
                            sqs = []
                            for (ctab, off) in ((crf, 0), (cjf, C2)):
                                n0, n1 = _f8_to_bf16(nbr[b, k, tl, pl.ds(off + cc * L, L)])
                                n2, n3 = _f8_to_bf16(nbr[b, k, tl, pl.ds(off + (cc + 1) * L, L)])
                                d0 = ctab[2 * cc] - n0
                                d1 = ctab[2 * cc + 1] - n1
                                d2 = ctab[2 * cc + 2] - n2
                                d3 = ctab[2 * cc + 3] - n3
                                sqs.append((d0 * d0 + d1 * d1) + (d2 * d2 + d3 * d3))
                            sq0, sq1 = plsc.unpack(sqs[0], format=plsc.PackFormat.INTERLEAVED)
                            arf = arf + sq0 + sq1
                            sq2, sq3 = plsc.unpack(sqs[1], format=plsc.PackFormat.INTERLEAVED)
                            ajf = ajf + sq2 + sq3
                        vrf = jnp.where(lanes == j, jnp.sum(arf), vrf)
                        vjf = jnp.where(lanes == j, jnp.sum(ajf), vjf)
                flat = obase + g * L
                row = lax.shift_right_logical(flat, 7)
                col = jnp.bitwise_and(flat, 127)
                drf_all[row, pl.ds(col, L)] = vrf
                djf_all[row, pl.ds(col, L)] = vjf
                return carry

            lax.fori_loop(0, Wp // L, gbody, 0)

        issue(0, 0)

        @pl.loop(0, NW, step=2)
        def _(w):
            for b in (0, 1):
                we = w + b

                @pl.when(we + 1 < NW)
                def _issue_next(we=we, b=b):
                    issue(we + 1, b ^ 1)

                wait(we, b)
                compute(we, b)

        orow = pl.multiple_of(pbase // 128, PPS // 128)
        pltpu.sync_copy(drf_all, drf_hbm.at[pl.ds(orow, PPS // 128), :])
        pltpu.sync_copy(djf_all, djf_hbm.at[pl.ds(orow, PPS // 128), :])

    return sck(tab_packed, kidx_nat)


def _tc_pack(rf, jf):
    BN, C = rf.shape
    C2 = C // 4
    R = 4096
    G = BN // R

    def body(rf_r, jf_r, o_r):
        for src, off in ((rf_r, 0), (jf_r, C2)):
            u = lax.bitcast_convert_type(
                src[...].astype(jnp.float8_e4m3fn), jnp.uint8).astype(jnp.int32)
            o_r[:, off:off + C2] = (u[:, :C2]
                                    | lax.shift_left(u[:, C2:2 * C2], 8)
                                    | lax.shift_left(u[:, 2 * C2:3 * C2], 16)
                                    | lax.shift_left(u[:, 3 * C2:], 24))

    return pl.pallas_call(
        body,
        grid=(G,),
        in_specs=[pl.BlockSpec((R, C), lambda i: (i, 0)),
                  pl.BlockSpec((R, C), lambda i: (i, 0))],
        out_specs=pl.BlockSpec((R, 2 * C2), lambda i: (i, 0)),
        out_shape=jax.ShapeDtypeStruct((BN, 2 * C2), jnp.int32),
    )(rf, jf)


def _tc_dense(rlog, alog, tgt2d, feat, prot):
    BN, NC = rlog.shape
    C = feat.shape[1]
    R = 2048
    G = BN // R

    def body(rlog_r, alog_r, tgt_r, feat_r, prot_r, out_r, acc):
        i = pl.program_id(0)

        @pl.when(i == 0)
        def _init():
            for j in range(8):
                acc[j] = 0.0

        tgt = tgt_r[...]
        valid = tgt != IGNORE_INDEX
        tgt0 = jnp.where(valid, tgt, 0)
        iota = lax.broadcasted_iota(jnp.int32, (R, NC), 1)
        onehot = iota == tgt0

        def ce_sum(lg):
            m = jnp.max(lg, axis=1, keepdims=True)
            l = lg - m
            lse = jnp.log(jnp.sum(jnp.exp(l), axis=1, keepdims=True))
            ltgt = jnp.sum(jnp.where(onehot, l, 0.0), axis=1, keepdims=True)
            return jnp.sum(jnp.where(valid, lse - ltgt, 0.0))

        s_main = ce_sum(rlog_r[...])
        s_aux = ce_sum(alog_r[...])
        n_valid = jnp.sum(valid.astype(jnp.float32))

        f = feat_r[...]
        p = prot_r[...]
        pn = p / jnp.maximum(jnp.sqrt(jnp.sum(p * p, axis=1, keepdims=True)), 1e-12)
        sim = lax.dot_general(f, pn, (((1,), (1,)), ((), ())),
                              preferred_element_type=jnp.float32)
        fnorm = jnp.maximum(jnp.sqrt(jnp.sum(f * f, axis=1, keepdims=True)), 1e-12)
        tsim = jnp.sum(jnp.where(onehot, sim, 0.0), axis=1, keepdims=True) / fnorm
        s_dist = jnp.sum(jnp.where(valid, 1.0 - tsim, 0.0))

        acc[0] += s_main
        acc[1] += s_aux
        acc[2] += s_dist
        acc[3] += n_valid
        for j in range(8):
            out_r[0, j] = acc[j]

    return pl.pallas_call(
        body,
        grid=(G,),
        in_specs=[
            pl.BlockSpec((R, NC), lambda i: (i, 0)),
            pl.BlockSpec((R, NC), lambda i: (i, 0)),
            pl.BlockSpec((R, 1), lambda i: (i, 0)),
            pl.BlockSpec((R, C), lambda i: (i, 0)),
            pl.BlockSpec((NC, C), lambda i: (0, 0)),
        ],
        out_specs=pl.BlockSpec((1, 8), lambda i: (0, 0),
                               memory_space=pltpu.SMEM),
        out_shape=jax.ShapeDtypeStruct((1, 8), jnp.float32),
        scratch_shapes=[pltpu.SMEM((8,), jnp.float32)],
    )(rlog, alog, tgt2d, feat, prot)


def _tc_combine(sums, aff, bdy, drf, djf, C, K, BN):
    PR = aff.shape[0]
    TPR = 128 // K
    R = 256
    G = PR // R

    def body(sums_r, aff_r, bdy_r, drf_r, djf_r, out_r, acc):
        i = pl.program_id(0)

        @pl.when(i == 0)
        def _init():
            for j in range(4):
                acc[j] = 0.0

        w = jnp.maximum(aff_r[...] - 0.5, 0.0)
        acc[0] += jnp.sum(w * drf_r[...])
        acc[1] += jnp.sum(w)

        lane = lax.broadcasted_iota(jnp.int32, (128, TPR), 0)
        tok = lax.broadcasted_iota(jnp.int32, (128, TPR), 1)
        m = (lane // K == tok).astype(jnp.float32)
        es = lax.dot_general(jnp.sqrt(djf_r[...]), m, (((1,), (0,)), ((), ())),
                             preferred_element_type=jnp.float32) / K
        tb = jax.nn.sigmoid((es - 0.15) * 20.0)
        x = bdy_r[...]
        bce = jnp.maximum(x, 0.0) - x * tb + jnp.log1p(jnp.exp(-jnp.abs(x)))
        acc[2] += jnp.sum(bce)

        denom = jnp.maximum(sums_r[0, 3], 1.0)
        loss = (L_MAIN * sums_r[0, 0] + L_AUX * sums_r[0, 1]) / denom
        loss += L_AFF * (acc[0] / (C ** 0.5)) / (acc[1] + 0.0001)
        loss += L_DIST * sums_r[0, 2] / denom
        loss += L_BDY * acc[2] / BN
        out_r[0, 0] = loss

    return pl.pallas_call(
        body,
        grid=(G,),
        in_specs=[
            pl.BlockSpec((1, 8), lambda i: (0, 0), memory_space=pltpu.SMEM),
            pl.BlockSpec((R, 128), lambda i: (i, 0)),
            pl.BlockSpec((R, TPR), lambda i: (i, 0)),
            pl.BlockSpec((R, 128), lambda i: (i, 0)),
            pl.BlockSpec((R, 128), lambda i: (i, 0)),
        ],
        out_specs=pl.BlockSpec((1, 1), lambda i: (0, 0),
                               memory_space=pltpu.SMEM),
        out_shape=jax.ShapeDtypeStruct((1, 1), jnp.float32),
        scratch_shapes=[pltpu.SMEM((4,), jnp.float32)],
    )(sums, aff, bdy, drf, djf)


def kernel(refined_logits, aux_logits, refined_feat, affinity, prototypes,
           input_jafar_feat, bdy_logits, target, k_idx):
    B, N, K = k_idx.shape
    C = refined_feat.shape[-1]
    BN = B * N
    P = BN * K

    kidx_nat = k_idx.transpose(0, 2, 1).reshape(B * K, N)
    rf_flat = refined_feat.reshape(BN, C)
    jf_flat = input_jafar_feat.reshape(BN, C)

    tab_p = _tc_pack(rf_flat, jf_flat)
    d_rf, d_jf = _sc_pair_d2(tab_p, kidx_nat)

    sums = _tc_dense(refined_logits, aux_logits, target.reshape(BN, 1),
                     rf_flat, prototypes)
    out = _tc_combine(sums, affinity.reshape(P // 128, 128),
                      bdy_logits.reshape(BN // 16, 16), d_rf, d_jf, C, K, BN)
    return out[0, 0]
